# baseline (speedup 1.0000x reference)
import jax
import jax.numpy as jnp
from jax.experimental import pallas as pl


_BM = 512


def _router_body(x_ref, w_ref, out_ref):
    out_ref[...] = jax.lax.dot_general(
        x_ref[...],
        w_ref[...],
        dimension_numbers=(((1,), (1,)), ((), ())),
        preferred_element_type=jnp.float32,
    )


def kernel(x, W):
    m, k = x.shape
    e = W.shape[0]
    grid = (m // _BM,)
    return pl.pallas_call(
        _router_body,
        grid=grid,
        in_specs=[
            pl.BlockSpec((_BM, k), lambda i: (i, 0)),
            pl.BlockSpec((e, k), lambda i: (0, 0)),
        ],
        out_specs=pl.BlockSpec((_BM, e), lambda i: (i, 0)),
        out_shape=jax.ShapeDtypeStruct((m, e), jnp.float32),
    )(x, W)


# speedup vs baseline: 1.0796x; 1.0796x over previous
import jax
import jax.numpy as jnp
from jax.experimental import pallas as pl


_BM = 2048


def _router_body(x_ref, w_ref, out_ref):
    out_ref[...] = jax.lax.dot_general(
        x_ref[...],
        w_ref[...],
        dimension_numbers=(((1,), (1,)), ((), ())),
        preferred_element_type=jnp.float32,
    )


def kernel(x, W):
    m, k = x.shape
    e = W.shape[0]
    grid = (m // _BM,)
    return pl.pallas_call(
        _router_body,
        grid=grid,
        in_specs=[
            pl.BlockSpec((_BM, k), lambda i: (i, 0)),
            pl.BlockSpec((e, k), lambda i: (0, 0)),
        ],
        out_specs=pl.BlockSpec((_BM, e), lambda i: (i, 0)),
        out_shape=jax.ShapeDtypeStruct((m, e), jnp.float32),
    )(x, W)


# speedup vs baseline: 1.0885x; 1.0082x over previous
import jax
import jax.numpy as jnp
from jax.experimental import pallas as pl
from jax.experimental.pallas import tpu as pltpu


_CHUNK = 512
_NBUF = 4


def _router_body(x_hbm, w_ref, out_ref, xbuf, sems):
    m = x_hbm.shape[0]
    nchunks = m // _CHUNK

    def _copy(chunk_idx, buf_idx):
        return pltpu.make_async_copy(
            x_hbm.at[pl.ds(chunk_idx * _CHUNK, _CHUNK), :],
            xbuf.at[buf_idx],
            sems.at[buf_idx],
        )

    for b in range(_NBUF):
        _copy(b, b).start()

    def step(i, carry):
        b = jax.lax.rem(i, _NBUF)
        _copy(i, b).wait()
        out_ref[pl.ds(i * _CHUNK, _CHUNK), :] = jax.lax.dot_general(
            xbuf[b],
            w_ref[...],
            dimension_numbers=(((1,), (1,)), ((), ())),
            preferred_element_type=jnp.float32,
        )

        @pl.when(i + _NBUF < nchunks)
        def _():
            _copy(i + _NBUF, b).start()

        return carry

    jax.lax.fori_loop(0, nchunks, step, 0)


def kernel(x, W):
    m, k = x.shape
    e = W.shape[0]
    return pl.pallas_call(
        _router_body,
        in_specs=[
            pl.BlockSpec(memory_space=pl.ANY),
            pl.BlockSpec((e, k), lambda: (0, 0)),
        ],
        out_specs=pl.BlockSpec((m, e), lambda: (0, 0)),
        out_shape=jax.ShapeDtypeStruct((m, e), jnp.float32),
        scratch_shapes=[
            pltpu.VMEM((_NBUF, _CHUNK, k), jnp.float32),
            pltpu.SemaphoreType.DMA((_NBUF,)),
        ],
    )(x, W)
